# Initial kernel scaffold; baseline (speedup 1.0000x reference)
import jax
import jax.numpy as jnp
from jax.experimental import pallas as pl


def kernel(mu, log_var, room_feat, rr_edge_feat, params, ff_edges, rr_edges, rf_edges, eps):
    raise NotImplementedError("write your pallas kernel here")


# speedup vs baseline: 46.3619x; 46.3619x over previous
import functools

import jax
import jax.numpy as jnp
from jax import lax
from jax.experimental import pallas as pl
from jax.experimental.pallas import tpu as pltpu
from jax.experimental.pallas import tpu_sc as plsc

NN = 10000
E_EDGES = 320000
NWORK = 32
EPAD = 327680
EW = EPAD // NWORK
NACC = 10112
NPS = NACC // 16
PW = 8
BR = 1000
NBLK = NN // BR

_f32 = jnp.float32
_i32 = jnp.int32


def _pad16(n):
    return (n + 15) // 16 * 16


@functools.lru_cache(maxsize=None)
def _edge_pass(d, dp):
    mesh = plsc.VectorSubcoreMesh(core_axis_name="c", subcore_axis_name="s",
                                  num_cores=2, num_subcores=16)
    wsz = _pad16(dp * d + d)
    CH = 1024
    NR = CH // 128
    NCHUNK = EW // CH
    scratch = [
        pltpu.VMEM((NN * d,), _f32),
        pltpu.VMEM((NN * d,), _f32),
        pltpu.VMEM((wsz,), _f32),
        pltpu.VMEM((NR, 128), _i32),
        pltpu.VMEM((NR, 128), _i32),
        pltpu.VMEM((max(dp, 1) * CH,), _f32),
        pltpu.VMEM((d * CH,), _f32),
        pltpu.VMEM((CH, PW), _f32),
        pltpu.VMEM((NPS, PW), _f32),
        pltpu.VMEM_SHARED((NACC, PW), _f32),
    ]
    out_type = [
        jax.ShapeDtypeStruct((d * EPAD,), _f32),
        jax.ShapeDtypeStruct((2, NACC, PW), _f32),
    ]

    @functools.partial(
        pl.kernel, out_type=out_type, mesh=mesh, scratch_types=scratch,
        name=f"edge_pass_d{d}_dp{dp}",
        compiler_params=pltpu.CompilerParams(needs_layout_passes=False,
                                             use_tc_tiling_on_sc=False))
    def kern(*refs):
        if dp > 0:
            (src_h, dst_h, ps_h, pd_h, ep_h, w_h, z_h, eo_h, acc_h,
             ps_v, pd_v, w_v, src_v, dst_v, ep_v, eo_v, pay_v, stg_v,
             acc_sh) = refs
        else:
            (src_h, dst_h, ps_h, pd_h, w_h, z_h, eo_h, acc_h,
             ps_v, pd_v, w_v, src_v, dst_v, ep_v, eo_v, pay_v, stg_v,
             acc_sh) = refs
            ep_h = None
        cid = lax.axis_index("c")
        sid = lax.axis_index("s")
        wid = sid * 2 + cid
        pltpu.sync_copy(ps_h, ps_v)
        pltpu.sync_copy(pd_h, pd_v)
        pltpu.sync_copy(w_h, w_v)
        wchunks = [w_v[pl.ds(j * 16, 16)] for j in range(wsz // 16)]
        wl = [wchunks[i // 16][i % 16] for i in range(dp * d + d)]
        soff = pl.multiple_of(sid * NPS, 8)
        pltpu.sync_copy(z_h.at[pl.ds(soff, NPS)], stg_v)
        pltpu.sync_copy(stg_v, acc_sh.at[pl.ds(soff, NPS)])
        plsc.subcore_barrier()

        def chunk(ci, carry):
            base = pl.multiple_of(wid * EW + ci * CH, CH)
            rbase = pl.multiple_of(base // 128, 8)
            pltpu.sync_copy(src_h.at[pl.ds(rbase, NR)], src_v)
            pltpu.sync_copy(dst_h.at[pl.ds(rbase, NR)], dst_v)
            if dp > 0:
                for k in range(dp):
                    pltpu.sync_copy(
                        ep_h.at[pl.ds(pl.multiple_of(k * EPAD + base, CH), CH)],
                        ep_v.at[pl.ds(k * CH, CH)])

            def irow(r, carry2):
                for g in range(8):
                    o = r * 128 + g * 16
                    ids = lax.iota(_i32, 16) + o
                    s16 = src_v[r, pl.ds(g * 16, 16)] * d
                    d16 = dst_v[r, pl.ds(g * 16, 16)] * d
                    eps_l = [ep_v[pl.ds(k * CH + o, 16)] for k in range(dp)]
                    logit = None
                    eo_l = []
                    for c in range(d):
                        acc = (plsc.load_gather(ps_v, [s16 + c])
                               + plsc.load_gather(pd_v, [d16 + c]))
                        for k in range(dp):
                            acc = acc + eps_l[k] * wl[k * d + c]
                        eo_c = jnp.maximum(acc, 0.0)
                        eo_v[pl.ds(c * CH + o, 16)] = eo_c
                        eo_l.append(eo_c)
                        t = eo_c * wl[dp * d + c]
                        logit = t if logit is None else logit + t
                    logit = jnp.where(logit > 0, logit, logit * 0.2)
                    ex = jnp.exp(logit)
                    plsc.store_scatter(pay_v, [ids, jnp.full((16,), 0, _i32)], ex)
                    for c in range(d):
                        plsc.store_scatter(
                            pay_v, [ids, jnp.full((16,), c + 1, _i32)],
                            ex * eo_l[c])
                pltpu.sync_copy(pay_v.at[pl.ds(r * 128, 128)],
                                acc_sh.at[dst_v.at[r]], add=True)
                return carry2

            lax.fori_loop(0, NR, irow, 0)
            for c in range(d):
                pltpu.sync_copy(
                    eo_v.at[pl.ds(c * CH, CH)],
                    eo_h.at[pl.ds(pl.multiple_of(c * EPAD + base, CH), CH)])
            return carry

        lax.fori_loop(0, NCHUNK, chunk, 0)
        plsc.subcore_barrier()
        pltpu.sync_copy(acc_sh.at[pl.ds(soff, NPS)], stg_v)
        pltpu.sync_copy(stg_v, acc_h.at[cid, pl.ds(soff, NPS)])

    return kern


def _full(spec_shape):
    return pl.BlockSpec(spec_shape, lambda i: (0,) * len(spec_shape))


def _rows(w):
    return pl.BlockSpec((BR, w), lambda i: (i, 0))


def _dot(a, b):
    return jnp.dot(a, b, preferred_element_type=_f32)


def _tc_pre1(mu, lv, eps, room, wffs, wffd, bff, wrrs, wrrd, brr, wrfs, wrfd, brf):
    def body(mu_r, lv_r, eps_r, room_r, wffs_r, wffd_r, bff_r, wrrs_r, wrrd_r,
             brr_r, wrfs_r, wrfd_r, brf_r,
             furn_o, pffs_o, pffd_o, prrs_o, prrd_o, prfs_o, prfd_o):
        furn = mu_r[...] + jnp.exp(0.5 * lv_r[...]) * eps_r[...]
        room = room_r[...]
        furn_o[...] = furn
        pffs_o[...] = _dot(furn, wffs_r[...])
        pffd_o[...] = _dot(furn, wffd_r[...]) + bff_r[...]
        prrs_o[...] = _dot(room, wrrs_r[...])
        prrd_o[...] = _dot(room, wrrd_r[...]) + brr_r[...]
        prfs_o[...] = _dot(room, wrfs_r[...])
        prfd_o[...] = _dot(furn, wrfd_r[...]) + brf_r[...]

    return pl.pallas_call(
        body,
        grid=(NBLK,),
        in_specs=[_rows(64), _rows(64), _rows(64), _rows(14),
                  _full((64, 3)), _full((64, 3)), _full((1, 3)),
                  _full((14, 4)), _full((14, 4)), _full((1, 4)),
                  _full((14, 5)), _full((64, 5)), _full((1, 5))],
        out_specs=[_rows(64), _rows(3), _rows(3), _rows(4), _rows(4),
                   _rows(5), _rows(5)],
        out_shape=[jax.ShapeDtypeStruct((NN, 64), _f32),
                   jax.ShapeDtypeStruct((NN, 3), _f32),
                   jax.ShapeDtypeStruct((NN, 3), _f32),
                   jax.ShapeDtypeStruct((NN, 4), _f32),
                   jax.ShapeDtypeStruct((NN, 4), _f32),
                   jax.ShapeDtypeStruct((NN, 5), _f32),
                   jax.ShapeDtypeStruct((NN, 5), _f32)],
    )(mu, lv, eps, room, wffs, wffd, bff, wrrs, wrrd, brr, wrfs, wrfd, brf)


def _aggs(a_r, d):
    u = a_r[0, :, 1:1 + d] + a_r[1, :, 1:1 + d]
    s = a_r[0, :, 0:1] + a_r[1, :, 0:1]
    return u / (s + 1e-9)


def _tc_mid(nf, nf2, room, furn, aff, arr, arf, self_room, wm_room, b_room,
            self_furn, wm_f1, wm_f2, b_furn,
            wffs, wffd, bff, wrrs, wrrd, brr, wrfs, wrfd, brf):
    def body(room_r, furn_r, aff_r, arr_r, arf_r, sr_r, wmr_r, br_r,
             sf_r, wf1_r, wf2_r, bf_r, wffs_r, wffd_r, bff_r,
             wrrs_r, wrrd_r, brr_r, wrfs_r, wrfd_r, brf_r,
             nroom_o, nfurn_o, pffs_o, pffd_o, prrs_o, prrd_o, prfs_o, prfd_o):
        agg_ff = _aggs(aff_r[...], 3)
        agg_rr = _aggs(arr_r[...], 4)
        agg_rf = _aggs(arf_r[...], 5)
        nroom = jnp.maximum(_dot(room_r[...], sr_r[...])
                            + _dot(agg_rr, wmr_r[...]) + br_r[...], 0.0)
        nfurn = jnp.maximum(_dot(furn_r[...], sf_r[...])
                            + _dot(agg_ff, wf1_r[...])
                            + _dot(agg_rf, wf2_r[...]) + bf_r[...], 0.0)
        nroom_o[...] = nroom
        nfurn_o[...] = nfurn
        pffs_o[...] = _dot(nfurn, wffs_r[...])
        pffd_o[...] = _dot(nfurn, wffd_r[...]) + bff_r[...]
        prrs_o[...] = _dot(nroom, wrrs_r[...])
        prrd_o[...] = _dot(nroom, wrrd_r[...]) + brr_r[...]
        prfs_o[...] = _dot(nroom, wrfs_r[...])
        prfd_o[...] = _dot(nfurn, wrfd_r[...]) + brf_r[...]

    acc_spec = pl.BlockSpec((2, BR, PW), lambda i: (0, i, 0))
    return pl.pallas_call(
        body,
        grid=(NBLK,),
        in_specs=[_rows(14), _rows(nf), acc_spec, acc_spec, acc_spec,
                  _full((14, 14)), _full((4, 14)), _full((1, 14)),
                  _full((nf, nf2)), _full((3, nf2)), _full((5, nf2)),
                  _full((1, nf2)),
                  _full((nf2, 3)), _full((nf2, 3)), _full((1, 3)),
                  _full((14, 4)), _full((14, 4)), _full((1, 4)),
                  _full((14, 5)), _full((nf2, 5)), _full((1, 5))],
        out_specs=[_rows(14), _rows(nf2), _rows(3), _rows(3), _rows(4),
                   _rows(4), _rows(5), _rows(5)],
        out_shape=[jax.ShapeDtypeStruct((NN, 14), _f32),
                   jax.ShapeDtypeStruct((NN, nf2), _f32),
                   jax.ShapeDtypeStruct((NN, 3), _f32),
                   jax.ShapeDtypeStruct((NN, 3), _f32),
                   jax.ShapeDtypeStruct((NN, 4), _f32),
                   jax.ShapeDtypeStruct((NN, 4), _f32),
                   jax.ShapeDtypeStruct((NN, 5), _f32),
                   jax.ShapeDtypeStruct((NN, 5), _f32)],
    )(room, furn, aff, arr, arf, self_room, wm_room, b_room, self_furn,
      wm_f1, wm_f2, b_furn, wffs, wffd, bff, wrrs, wrrd, brr, wrfs, wrfd, brf)


def _tc_fin(nf, nf2, furn, aff, arf, self_furn, wm_f1, wm_f2, b_furn):
    def body(furn_r, aff_r, arf_r, sf_r, wf1_r, wf2_r, bf_r, nfurn_o):
        agg_ff = _aggs(aff_r[...], 3)
        agg_rf = _aggs(arf_r[...], 5)
        nfurn_o[...] = jnp.maximum(
            _dot(furn_r[...], sf_r[...]) + _dot(agg_ff, wf1_r[...])
            + _dot(agg_rf, wf2_r[...]) + bf_r[...], 0.0)

    acc_spec = pl.BlockSpec((2, BR, PW), lambda i: (0, i, 0))
    return pl.pallas_call(
        body,
        grid=(NBLK,),
        in_specs=[_rows(nf), acc_spec, acc_spec,
                  _full((nf, nf2)), _full((3, nf2)), _full((5, nf2)),
                  _full((1, nf2))],
        out_specs=[_rows(nf2)],
        out_shape=[jax.ShapeDtypeStruct((NN, nf2), _f32)],
    )(furn, aff, arf, self_furn, wm_f1, wm_f2, b_furn)[0]


def _prep_edges(edges):
    pad = EPAD - E_EDGES
    src = jnp.concatenate([edges[0], jnp.zeros((pad,), _i32)])
    dst = jnp.concatenate([edges[1], jnp.full((pad,), NN, _i32)])
    return src.reshape(EPAD // 128, 128), dst.reshape(EPAD // 128, 128)


def _wvec(wee, a):
    flat = jnp.concatenate([wee.reshape(-1), a]) if wee is not None else a
    return jnp.pad(flat, (0, _pad16(flat.shape[0]) - flat.shape[0]))


def kernel(mu, log_var, room_feat, rr_edge_feat, params, ff_edges, rr_edges,
           rf_edges, eps):
    nin_f = [64, 128, 256]
    nout_f = [128, 256, 1040]

    sff, dff = _prep_edges(ff_edges)
    srr, drr = _prep_edges(rr_edges)
    srf, drf = _prep_edges(rf_edges)
    zeros = jnp.zeros((NACC, PW), _f32)

    def split_w(l, et):
        p = params["L%d" % (l + 1)]
        We, be, a = p[et]["We"], p[et]["be"], p[et]["a"]
        ns = 14 if et in ("rr", "rf") else nin_f[l]
        nd = 14 if et == "rr" else nin_f[l]
        return We[:ns], We[ns:ns + nd], We[ns + nd:], be, a

    ws, wd, we, be, aff1 = split_w(0, "ff")
    wffs1, wffd1, bff1 = ws + we[:64], wd + we[64:], be[None, :]
    ws, wd, wee_rr1, be, arr1 = split_w(0, "rr")
    wrrs1, wrrd1, brr1 = ws, wd, be[None, :]
    ws, wd, we, be, arf1 = split_w(0, "rf")
    wrfs1, wrfd1, brf1 = ws + we[:14], wd + we[14:], be[None, :]

    furn, pffs, pffd, prrs, prrd, prfs, prfd = _tc_pre1(
        mu, log_var, eps, room_feat, wffs1, wffd1, bff1, wrrs1, wrrd1, brr1,
        wrfs1, wrfd1, brf1)

    ep_rr = jnp.pad(rr_edge_feat.T, ((0, 0), (0, EPAD - E_EDGES))).reshape(-1)
    h_room, h_furn = room_feat, furn
    ep = {"ff": None, "rr": ep_rr, "rf": None}
    avec = {"ff": aff1, "rr": arr1, "rf": arf1}
    wee = {"ff": None, "rr": wee_rr1, "rf": None}

    out = None
    for l in range(3):
        accs = {}
        new_ep = {}
        for et, dd in (("ff", 3), ("rr", 4), ("rf", 5)):
            s_, d_ = {"ff": (sff, dff), "rr": (srr, drr), "rf": (srf, drf)}[et]
            ps_, pd_ = {"ff": (pffs, pffd), "rr": (prrs, prrd),
                        "rf": (prfs, prfd)}[et]
            dp = 0 if ep[et] is None else dd
            w_ = _wvec(wee[et], avec[et])
            psf, pdf = ps_.reshape(-1), pd_.reshape(-1)
            if dp > 0:
                eo, acc = _edge_pass(dd, dp)(s_, d_, psf, pdf, ep[et], w_, zeros)
            else:
                eo, acc = _edge_pass(dd, 0)(s_, d_, psf, pdf, w_, zeros)
            new_ep[et] = eo
            accs[et] = acc[:, :NN, :]
        ep = new_ep

        p = params["L%d" % (l + 1)]
        wm_f1 = p["Wm_furn"][:3]
        wm_f2 = p["Wm_furn"][3:]
        if l < 2:
            ws, wd, wee_ff, be, a_ff = split_w(l + 1, "ff")
            wffs_, wffd_, bff_ = ws, wd, be[None, :]
            ws, wd, wee_rr, be, a_rr = split_w(l + 1, "rr")
            wrrs_, wrrd_, brr_ = ws, wd, be[None, :]
            ws, wd, wee_rf, be, a_rf = split_w(l + 1, "rf")
            wrfs_, wrfd_, brf_ = ws, wd, be[None, :]
            (h_room, h_furn, pffs, pffd, prrs, prrd, prfs, prfd) = _tc_mid(
                nin_f[l], nout_f[l], h_room, h_furn,
                accs["ff"], accs["rr"], accs["rf"],
                p["self_room"], p["Wm_room"], p["b_room"][None, :],
                p["self_furn"], wm_f1, wm_f2, p["b_furn"][None, :],
                wffs_, wffd_, bff_, wrrs_, wrrd_, brr_, wrfs_, wrfd_, brf_)
            avec = {"ff": a_ff, "rr": a_rr, "rf": a_rf}
            wee = {"ff": wee_ff, "rr": wee_rr, "rf": wee_rf}
        else:
            out = _tc_fin(nin_f[l], nout_f[l], h_furn, accs["ff"], accs["rf"],
                          p["self_furn"], wm_f1, wm_f2, p["b_furn"][None, :])
    return out


# speedup vs baseline: 57.8852x; 1.2486x over previous
import functools

import jax
import jax.numpy as jnp
from jax import lax
from jax.experimental import pallas as pl
from jax.experimental.pallas import tpu as pltpu
from jax.experimental.pallas import tpu_sc as plsc

NN = 10000
E_EDGES = 320000
NWORK = 32
EPAD = 327680
EW = EPAD // NWORK
NACC = 10112
NPS = NACC // 16
PW = 8
BR = 1000
NBLK = NN // BR

_f32 = jnp.float32
_i32 = jnp.int32


def _pad16(n):
    return (n + 15) // 16 * 16


@functools.lru_cache(maxsize=None)
def _edge_pass(d, dp, last):
    mesh = plsc.VectorSubcoreMesh(core_axis_name="c", subcore_axis_name="s",
                                  num_cores=2, num_subcores=16)
    wsz = _pad16(dp * d + d)
    CH = 1024
    NR = CH // 128
    NCHUNK = EW // CH
    scratch = [
        pltpu.VMEM((NN * d,), _f32),
        pltpu.VMEM((NN * d,), _f32),
        pltpu.VMEM((wsz,), _f32),
        pltpu.VMEM((NR, 128), _i32),
        pltpu.VMEM((NR, 128), _i32),
        pltpu.VMEM((max(dp, 1) * CH,), _f32),
        pltpu.VMEM((d * CH,), _f32),
        pltpu.VMEM((CH, PW), _f32),
        pltpu.VMEM((NPS, PW), _f32),
        pltpu.VMEM_SHARED((NACC, PW), _f32),
        pltpu.SemaphoreType.DMA,
        pltpu.SemaphoreType.DMA,
    ]
    out_type = [jax.ShapeDtypeStruct((2, NACC, PW), _f32)]
    if not last:
        out_type = [jax.ShapeDtypeStruct((d * EPAD,), _f32)] + out_type

    @functools.partial(
        pl.kernel, out_type=out_type, mesh=mesh, scratch_types=scratch,
        name=f"edge_pass_d{d}_dp{dp}_{int(last)}",
        compiler_params=pltpu.CompilerParams(needs_layout_passes=False,
                                             use_tc_tiling_on_sc=False))
    def kern(*refs):
        ins, refs = refs[:5 + (dp > 0)], refs[5 + (dp > 0):]
        if dp > 0:
            src_h, dst_h, ps_h, pd_h, ep_h, w_h = ins
        else:
            (src_h, dst_h, ps_h, pd_h, w_h), ep_h = ins, None
        z_h = refs[0]
        refs = refs[1:]
        if last:
            (acc_h,), refs = refs[:1], refs[1:]
            eo_h = None
        else:
            (eo_h, acc_h), refs = refs[:2], refs[2:]
        (ps_v, pd_v, w_v, src_v, dst_v, ep_v, eo_v, pay_v, stg_v, acc_sh,
         in_sem, out_sem) = refs
        cid = lax.axis_index("c")
        sid = lax.axis_index("s")
        wid = sid * 2 + cid
        pltpu.sync_copy(ps_h, ps_v)
        pltpu.sync_copy(pd_h, pd_v)
        pltpu.sync_copy(w_h, w_v)
        wchunks = [w_v[pl.ds(j * 16, 16)] for j in range(wsz // 16)]
        wl = [wchunks[i // 16][i % 16] for i in range(dp * d + d)]
        soff = pl.multiple_of(sid * NPS, 8)
        pltpu.sync_copy(z_h.at[pl.ds(soff, NPS)], stg_v)
        pltpu.sync_copy(stg_v, acc_sh.at[pl.ds(soff, NPS)])
        plsc.subcore_barrier()

        def chunk(ci, carry):
            base = pl.multiple_of(wid * EW + ci * CH, CH)
            rbase = pl.multiple_of(base // 128, 8)
            hs = [pltpu.async_copy(src_h.at[pl.ds(rbase, NR)], src_v, in_sem),
                  pltpu.async_copy(dst_h.at[pl.ds(rbase, NR)], dst_v, in_sem)]
            for k in range(dp):
                hs.append(pltpu.async_copy(
                    ep_h.at[pl.ds(pl.multiple_of(k * EPAD + base, CH), CH)],
                    ep_v.at[pl.ds(k * CH, CH)], in_sem))
            for h in hs:
                h.wait()

            def irow(r, carry2):
                for g in range(8):
                    o = r * 128 + g * 16
                    ids = lax.iota(_i32, 16) + o
                    s16 = src_v[r, pl.ds(g * 16, 16)] * d
                    d16 = dst_v[r, pl.ds(g * 16, 16)] * d
                    eps_l = [ep_v[pl.ds(k * CH + o, 16)] for k in range(dp)]
                    logit = None
                    eo_l = []
                    for c in range(d):
                        acc = (plsc.load_gather(ps_v, [s16 + c])
                               + plsc.load_gather(pd_v, [d16 + c]))
                        for k in range(dp):
                            acc = acc + eps_l[k] * wl[k * d + c]
                        eo_c = jnp.maximum(acc, 0.0)
                        if not last:
                            eo_v[pl.ds(c * CH + o, 16)] = eo_c
                        eo_l.append(eo_c)
                        t = eo_c * wl[dp * d + c]
                        logit = t if logit is None else logit + t
                    logit = jnp.where(logit > 0, logit, logit * 0.2)
                    ex = jnp.exp(logit)
                    plsc.store_scatter(pay_v, [ids, jnp.full((16,), 0, _i32)], ex)
                    for c in range(d):
                        plsc.store_scatter(
                            pay_v, [ids, jnp.full((16,), c + 1, _i32)],
                            ex * eo_l[c])
                pltpu.sync_copy(pay_v.at[pl.ds(r * 128, 128)],
                                acc_sh.at[dst_v.at[r]], add=True)
                return carry2

            lax.fori_loop(0, NR, irow, 0)
            if not last:
                ho = [pltpu.async_copy(
                    eo_v.at[pl.ds(c * CH, CH)],
                    eo_h.at[pl.ds(pl.multiple_of(c * EPAD + base, CH), CH)],
                    out_sem) for c in range(d)]
                for h in ho:
                    h.wait()
            return carry

        lax.fori_loop(0, NCHUNK, chunk, 0)
        plsc.subcore_barrier()
        pltpu.sync_copy(acc_sh.at[pl.ds(soff, NPS)], stg_v)
        pltpu.sync_copy(stg_v, acc_h.at[cid, pl.ds(soff, NPS)])

    return kern


def _full(spec_shape):
    return pl.BlockSpec(spec_shape, lambda i: (0,) * len(spec_shape))


def _rows(w):
    return pl.BlockSpec((BR, w), lambda i: (i, 0))


def _dot(a, b):
    return jnp.dot(a, b, preferred_element_type=_f32)


def _tc_pre1(mu, lv, eps, room, wffs, wffd, bff, wrrs, wrrd, brr, wrfs, wrfd, brf):
    def body(mu_r, lv_r, eps_r, room_r, wffs_r, wffd_r, bff_r, wrrs_r, wrrd_r,
             brr_r, wrfs_r, wrfd_r, brf_r,
             furn_o, pffs_o, pffd_o, prrs_o, prrd_o, prfs_o, prfd_o):
        furn = mu_r[...] + jnp.exp(0.5 * lv_r[...]) * eps_r[...]
        room = room_r[...]
        furn_o[...] = furn
        pffs_o[...] = _dot(furn, wffs_r[...])
        pffd_o[...] = _dot(furn, wffd_r[...]) + bff_r[...]
        prrs_o[...] = _dot(room, wrrs_r[...])
        prrd_o[...] = _dot(room, wrrd_r[...]) + brr_r[...]
        prfs_o[...] = _dot(room, wrfs_r[...])
        prfd_o[...] = _dot(furn, wrfd_r[...]) + brf_r[...]

    return pl.pallas_call(
        body,
        grid=(NBLK,),
        in_specs=[_rows(64), _rows(64), _rows(64), _rows(14),
                  _full((64, 3)), _full((64, 3)), _full((1, 3)),
                  _full((14, 4)), _full((14, 4)), _full((1, 4)),
                  _full((14, 5)), _full((64, 5)), _full((1, 5))],
        out_specs=[_rows(64), _rows(3), _rows(3), _rows(4), _rows(4),
                   _rows(5), _rows(5)],
        out_shape=[jax.ShapeDtypeStruct((NN, 64), _f32),
                   jax.ShapeDtypeStruct((NN, 3), _f32),
                   jax.ShapeDtypeStruct((NN, 3), _f32),
                   jax.ShapeDtypeStruct((NN, 4), _f32),
                   jax.ShapeDtypeStruct((NN, 4), _f32),
                   jax.ShapeDtypeStruct((NN, 5), _f32),
                   jax.ShapeDtypeStruct((NN, 5), _f32)],
    )(mu, lv, eps, room, wffs, wffd, bff, wrrs, wrrd, brr, wrfs, wrfd, brf)


def _aggs(a_r, d):
    u = a_r[0, :, 1:1 + d] + a_r[1, :, 1:1 + d]
    s = a_r[0, :, 0:1] + a_r[1, :, 0:1]
    return u / (s + 1e-9)


def _tc_mid(nf, nf2, room, furn, aff, arr, arf, self_room, wm_room, b_room,
            self_furn, wm_f1, wm_f2, b_furn,
            wffs, wffd, bff, wrrs, wrrd, brr, wrfs, wrfd, brf):
    def body(room_r, furn_r, aff_r, arr_r, arf_r, sr_r, wmr_r, br_r,
             sf_r, wf1_r, wf2_r, bf_r, wffs_r, wffd_r, bff_r,
             wrrs_r, wrrd_r, brr_r, wrfs_r, wrfd_r, brf_r,
             nroom_o, nfurn_o, pffs_o, pffd_o, prrs_o, prrd_o, prfs_o, prfd_o):
        agg_ff = _aggs(aff_r[...], 3)
        agg_rr = _aggs(arr_r[...], 4)
        agg_rf = _aggs(arf_r[...], 5)
        nroom = jnp.maximum(_dot(room_r[...], sr_r[...])
                            + _dot(agg_rr, wmr_r[...]) + br_r[...], 0.0)
        nfurn = jnp.maximum(_dot(furn_r[...], sf_r[...])
                            + _dot(agg_ff, wf1_r[...])
                            + _dot(agg_rf, wf2_r[...]) + bf_r[...], 0.0)
        nroom_o[...] = nroom
        nfurn_o[...] = nfurn
        pffs_o[...] = _dot(nfurn, wffs_r[...])
        pffd_o[...] = _dot(nfurn, wffd_r[...]) + bff_r[...]
        prrs_o[...] = _dot(nroom, wrrs_r[...])
        prrd_o[...] = _dot(nroom, wrrd_r[...]) + brr_r[...]
        prfs_o[...] = _dot(nroom, wrfs_r[...])
        prfd_o[...] = _dot(nfurn, wrfd_r[...]) + brf_r[...]

    acc_spec = pl.BlockSpec((2, BR, PW), lambda i: (0, i, 0))
    return pl.pallas_call(
        body,
        grid=(NBLK,),
        in_specs=[_rows(14), _rows(nf), acc_spec, acc_spec, acc_spec,
                  _full((14, 14)), _full((4, 14)), _full((1, 14)),
                  _full((nf, nf2)), _full((3, nf2)), _full((5, nf2)),
                  _full((1, nf2)),
                  _full((nf2, 3)), _full((nf2, 3)), _full((1, 3)),
                  _full((14, 4)), _full((14, 4)), _full((1, 4)),
                  _full((14, 5)), _full((nf2, 5)), _full((1, 5))],
        out_specs=[_rows(14), _rows(nf2), _rows(3), _rows(3), _rows(4),
                   _rows(4), _rows(5), _rows(5)],
        out_shape=[jax.ShapeDtypeStruct((NN, 14), _f32),
                   jax.ShapeDtypeStruct((NN, nf2), _f32),
                   jax.ShapeDtypeStruct((NN, 3), _f32),
                   jax.ShapeDtypeStruct((NN, 3), _f32),
                   jax.ShapeDtypeStruct((NN, 4), _f32),
                   jax.ShapeDtypeStruct((NN, 4), _f32),
                   jax.ShapeDtypeStruct((NN, 5), _f32),
                   jax.ShapeDtypeStruct((NN, 5), _f32)],
    )(room, furn, aff, arr, arf, self_room, wm_room, b_room, self_furn,
      wm_f1, wm_f2, b_furn, wffs, wffd, bff, wrrs, wrrd, brr, wrfs, wrfd, brf)


def _tc_fin(nf, nf2, furn, aff, arf, self_furn, wm_f1, wm_f2, b_furn):
    def body(furn_r, aff_r, arf_r, sf_r, wf1_r, wf2_r, bf_r, nfurn_o):
        agg_ff = _aggs(aff_r[...], 3)
        agg_rf = _aggs(arf_r[...], 5)
        nfurn_o[...] = jnp.maximum(
            _dot(furn_r[...], sf_r[...]) + _dot(agg_ff, wf1_r[...])
            + _dot(agg_rf, wf2_r[...]) + bf_r[...], 0.0)

    acc_spec = pl.BlockSpec((2, BR, PW), lambda i: (0, i, 0))
    return pl.pallas_call(
        body,
        grid=(NBLK,),
        in_specs=[_rows(nf), acc_spec, acc_spec,
                  _full((nf, nf2)), _full((3, nf2)), _full((5, nf2)),
                  _full((1, nf2))],
        out_specs=[_rows(nf2)],
        out_shape=[jax.ShapeDtypeStruct((NN, nf2), _f32)],
    )(furn, aff, arf, self_furn, wm_f1, wm_f2, b_furn)[0]


def _prep_edges(edges):
    pad = EPAD - E_EDGES
    src = jnp.concatenate([edges[0], jnp.zeros((pad,), _i32)])
    dst = jnp.concatenate([edges[1], jnp.full((pad,), NN, _i32)])
    return src.reshape(EPAD // 128, 128), dst.reshape(EPAD // 128, 128)


def _wvec(wee, a):
    flat = jnp.concatenate([wee.reshape(-1), a]) if wee is not None else a
    return jnp.pad(flat, (0, _pad16(flat.shape[0]) - flat.shape[0]))


def kernel(mu, log_var, room_feat, rr_edge_feat, params, ff_edges, rr_edges,
           rf_edges, eps):
    nin_f = [64, 128, 256]
    nout_f = [128, 256, 1040]

    sff, dff = _prep_edges(ff_edges)
    srr, drr = _prep_edges(rr_edges)
    srf, drf = _prep_edges(rf_edges)
    zeros = jnp.zeros((NACC, PW), _f32)

    def split_w(l, et):
        p = params["L%d" % (l + 1)]
        We, be, a = p[et]["We"], p[et]["be"], p[et]["a"]
        ns = 14 if et in ("rr", "rf") else nin_f[l]
        nd = 14 if et == "rr" else nin_f[l]
        return We[:ns], We[ns:ns + nd], We[ns + nd:], be, a

    ws, wd, we, be, aff1 = split_w(0, "ff")
    wffs1, wffd1, bff1 = ws + we[:64], wd + we[64:], be[None, :]
    ws, wd, wee_rr1, be, arr1 = split_w(0, "rr")
    wrrs1, wrrd1, brr1 = ws, wd, be[None, :]
    ws, wd, we, be, arf1 = split_w(0, "rf")
    wrfs1, wrfd1, brf1 = ws + we[:14], wd + we[14:], be[None, :]

    furn, pffs, pffd, prrs, prrd, prfs, prfd = _tc_pre1(
        mu, log_var, eps, room_feat, wffs1, wffd1, bff1, wrrs1, wrrd1, brr1,
        wrfs1, wrfd1, brf1)

    ep_rr = jnp.pad(rr_edge_feat.T, ((0, 0), (0, EPAD - E_EDGES))).reshape(-1)
    h_room, h_furn = room_feat, furn
    ep = {"ff": None, "rr": ep_rr, "rf": None}
    avec = {"ff": aff1, "rr": arr1, "rf": arf1}
    wee = {"ff": None, "rr": wee_rr1, "rf": None}

    out = None
    for l in range(3):
        accs = {}
        new_ep = {}
        for et, dd in (("ff", 3), ("rr", 4), ("rf", 5)):
            s_, d_ = {"ff": (sff, dff), "rr": (srr, drr), "rf": (srf, drf)}[et]
            ps_, pd_ = {"ff": (pffs, pffd), "rr": (prrs, prrd),
                        "rf": (prfs, prfd)}[et]
            dp = 0 if ep[et] is None else dd
            w_ = _wvec(wee[et], avec[et])
            psf, pdf = ps_.reshape(-1), pd_.reshape(-1)
            last = l == 2
            if dp > 0:
                res = _edge_pass(dd, dp, last)(s_, d_, psf, pdf, ep[et], w_,
                                               zeros)
            else:
                res = _edge_pass(dd, 0, last)(s_, d_, psf, pdf, w_, zeros)
            if last:
                (acc,) = res
                new_ep[et] = None
            else:
                eo, acc = res
                new_ep[et] = eo
            accs[et] = acc[:, :NN, :]
        ep = new_ep

        p = params["L%d" % (l + 1)]
        wm_f1 = p["Wm_furn"][:3]
        wm_f2 = p["Wm_furn"][3:]
        if l < 2:
            ws, wd, wee_ff, be, a_ff = split_w(l + 1, "ff")
            wffs_, wffd_, bff_ = ws, wd, be[None, :]
            ws, wd, wee_rr, be, a_rr = split_w(l + 1, "rr")
            wrrs_, wrrd_, brr_ = ws, wd, be[None, :]
            ws, wd, wee_rf, be, a_rf = split_w(l + 1, "rf")
            wrfs_, wrfd_, brf_ = ws, wd, be[None, :]
            (h_room, h_furn, pffs, pffd, prrs, prrd, prfs, prfd) = _tc_mid(
                nin_f[l], nout_f[l], h_room, h_furn,
                accs["ff"], accs["rr"], accs["rf"],
                p["self_room"], p["Wm_room"], p["b_room"][None, :],
                p["self_furn"], wm_f1, wm_f2, p["b_furn"][None, :],
                wffs_, wffd_, bff_, wrrs_, wrrd_, brr_, wrfs_, wrfd_, brf_)
            avec = {"ff": a_ff, "rr": a_rr, "rf": a_rf}
            wee = {"ff": wee_ff, "rr": wee_rr, "rf": wee_rf}
        else:
            out = _tc_fin(nin_f[l], nout_f[l], h_furn, accs["ff"], accs["rf"],
                          p["self_furn"], wm_f1, wm_f2, p["b_furn"][None, :])
    return out
